# baseline (speedup 1.0000x reference)
import functools

import jax
import jax.numpy as jnp
from jax import lax
from jax.experimental import pallas as pl
from jax.experimental.pallas import tpu as pltpu
from jax.experimental.pallas import tpu_sc as plsc

_NC, _NS = 2, 16
_NW = _NC * _NS
_CHUNK = 32
_NBUF = 2


def _make_sc_copy(max_pos, emb_dim, dtype):
    rows_per_w = max_pos // _NW
    n_chunks = rows_per_w // _CHUNK
    mesh = plsc.VectorSubcoreMesh(core_axis_name="c", subcore_axis_name="s")

    @functools.partial(
        pl.kernel,
        mesh=mesh,
        out_type=jax.ShapeDtypeStruct((max_pos, emb_dim), dtype),
        scratch_types=[
            pltpu.VMEM_SHARED((_NS, _NBUF, _CHUNK, emb_dim), dtype),
            pltpu.SemaphoreType.DMA((_NBUF,)),
            pltpu.SemaphoreType.DMA((_NBUF,)),
        ],
    )
    def sc_copy(pe_hbm, out_hbm, buf, in_sems, out_sems):
        sid = lax.axis_index("s")
        wid = sid * _NC + lax.axis_index("c")
        base = wid * rows_per_w

        def src(g):
            return pe_hbm.at[pl.ds(base + g * _CHUNK, _CHUNK)]

        def dst(g):
            return out_hbm.at[pl.ds(base + g * _CHUNK, _CHUNK)]

        ins = {}
        outs = {}
        for b in range(min(_NBUF, n_chunks)):
            ins[b] = pltpu.async_copy(src(b), buf.at[sid, b], in_sems.at[b])
        for g in range(n_chunks):
            b = g % _NBUF
            ins[g].wait()
            outs[g] = pltpu.async_copy(buf.at[sid, b], dst(g), out_sems.at[b])
            ng = g + _NBUF
            if ng < n_chunks:
                outs[g].wait()
                ins[ng] = pltpu.async_copy(src(ng), buf.at[sid, b], in_sems.at[b])
        for g in range(max(0, n_chunks - _NBUF), n_chunks):
            outs[g].wait()

    return sc_copy


def kernel(x, pe_table):
    del x
    max_pos, emb_dim = pe_table.shape
    out = _make_sc_copy(max_pos, emb_dim, pe_table.dtype)(pe_table)
    return out[None]


# speedup vs baseline: 1.0387x; 1.0387x over previous
import functools

import jax
import jax.numpy as jnp
from jax import lax
from jax.experimental import pallas as pl
from jax.experimental.pallas import tpu as pltpu
from jax.experimental.pallas import tpu_sc as plsc

_NC, _NS = 2, 16
_NW = _NC * _NS
_CHUNK = 32
_NBUF = 3
_NBUF_SH = 2


def _make_sc_copy(max_pos, emb_dim, dtype):
    rows_per_w = max_pos // _NW
    n_chunks = rows_per_w // _CHUNK
    mesh = plsc.VectorSubcoreMesh(core_axis_name="c", subcore_axis_name="s")

    @functools.partial(
        pl.kernel,
        mesh=mesh,
        out_type=jax.ShapeDtypeStruct((max_pos, emb_dim), dtype),
        scratch_types=[
            pltpu.VMEM((_NBUF, _CHUNK, emb_dim), dtype),
            pltpu.VMEM_SHARED((_NS // 2, _NBUF_SH, _CHUNK, emb_dim), dtype),
            pltpu.SemaphoreType.DMA((_NBUF,)),
            pltpu.SemaphoreType.DMA((_NBUF,)),
        ],
    )
    def sc_copy(pe_hbm, out_hbm, tbuf, sbuf, in_sems, out_sems):
        sid = lax.axis_index("s")
        wid = sid * _NC + lax.axis_index("c")
        base = wid * rows_per_w

        def src(g):
            return pe_hbm.at[pl.ds(base + g * _CHUNK, _CHUNK)]

        def dst(g):
            return out_hbm.at[pl.ds(base + g * _CHUNK, _CHUNK)]

        def ring(buf_at, nbuf):
            ins = {}
            outs = {}
            for b in range(min(nbuf, n_chunks)):
                ins[b] = pltpu.async_copy(src(b), buf_at(b), in_sems.at[b])
            for g in range(n_chunks):
                b = g % nbuf
                ins[g].wait()
                outs[g] = pltpu.async_copy(buf_at(b), dst(g), out_sems.at[b])
                ng = g + nbuf
                if ng < n_chunks:
                    outs[g].wait()
                    ins[ng] = pltpu.async_copy(src(ng), buf_at(b), in_sems.at[b])
            for g in range(max(0, n_chunks - nbuf), n_chunks):
                outs[g].wait()

        @pl.when(sid % 2 == 0)
        def _tile_path():
            ring(lambda b: tbuf.at[b], _NBUF)

        @pl.when(sid % 2 == 1)
        def _spmem_path():
            ring(lambda b: sbuf.at[sid // 2, b], _NBUF_SH)

    return sc_copy


def kernel(x, pe_table):
    del x
    max_pos, emb_dim = pe_table.shape
    out = _make_sc_copy(max_pos, emb_dim, pe_table.dtype)(pe_table)
    return out[None]
